# Initial kernel scaffold; baseline (speedup 1.0000x reference)
import jax
import jax.numpy as jnp
from jax.experimental import pallas as pl


def kernel(emg_latent, emg_parallel_latent):
    raise NotImplementedError("write your pallas kernel here")


# speedup vs baseline: 8.7910x; 8.7910x over previous
import functools

import jax
import jax.numpy as jnp
from jax.experimental import pallas as pl
from jax.experimental.pallas import tpu as pltpu

_EPS = 1e-8


def _normalize_body(a_ref, b_ref, o_ref, *, f):
    a = a_ref[...]
    b = b_ref[...]
    ss = (jnp.sum(a * a, axis=1, keepdims=True)
          + jnp.sum(b * b, axis=1, keepdims=True))
    inv = 1.0 / jnp.maximum(jnp.sqrt(ss), _EPS)
    o_ref[:, :f] = (a * inv).astype(o_ref.dtype)
    o_ref[:, f:] = (b * inv).astype(o_ref.dtype)


def _nn_body(x_ref, y_ref, logd_ref, sum_ref, acc_ref, *, bm, bn, cn, nj, d):
    i = pl.program_id(0)
    j = pl.program_id(1)

    @pl.when(j == 0)
    def _():
        acc_ref[...] = jnp.full_like(acc_ref, -3.0)

    xi = x_ref[...]

    def accum(mask_diag):
        m = None
        for c in range(bn // cn):
            yc = y_ref[c * cn:(c + 1) * cn, :]
            sub = jax.lax.dot_general(
                xi, yc, (((1,), (1,)), ((), ())),
                preferred_element_type=jnp.float32)
            if mask_diag:
                rows = jax.lax.broadcasted_iota(jnp.int32, (bm, cn), 0)
                cols = jax.lax.broadcasted_iota(jnp.int32, (bm, cn), 1)
                sub = jnp.where(rows == cols + c * cn, -1.0, sub)
            for l in range(cn // 128):
                piece = sub[:, l * 128:(l + 1) * 128]
                m = piece if m is None else jnp.maximum(m, piece)
        acc_ref[...] = jnp.maximum(acc_ref[...], m)

    pl.when(i == j)(lambda: accum(True))
    pl.when(i != j)(lambda: accum(False))

    @pl.when(j == nj - 1)
    def _():
        mrow = jnp.max(acc_ref[...], axis=1)
        d2 = jnp.maximum(2.0 - 2.0 * mrow, 0.0)
        dist = jnp.sqrt(d2 + d * (_EPS * _EPS))
        logd = jnp.log(dist + _EPS)
        logd_ref[0, 0, :] = logd
        sum_ref[0, 0, :] = jnp.full((128,), jnp.sum(logd), jnp.float32)


def kernel(emg_latent, emg_parallel_latent):
    n, f = emg_latent.shape
    d = 2 * f

    rb = min(n, 2048)
    xn = pl.pallas_call(
        functools.partial(_normalize_body, f=f),
        grid=(n // rb,),
        in_specs=[pl.BlockSpec((rb, f), lambda r: (r, 0)),
                  pl.BlockSpec((rb, f), lambda r: (r, 0))],
        out_specs=pl.BlockSpec((rb, d), lambda r: (r, 0)),
        out_shape=jax.ShapeDtypeStruct((n, d), jnp.bfloat16),
        compiler_params=pltpu.CompilerParams(
            dimension_semantics=("parallel",)),
        name="koleo_normalize",
    )(emg_latent, emg_parallel_latent)

    bm = min(n, 1024)
    bn = min(n, 1024)
    cn = min(bn, 256)
    ni = n // bm
    nj = n // bn

    logd, sums = pl.pallas_call(
        functools.partial(_nn_body, bm=bm, bn=bn, cn=cn, nj=nj, d=d),
        grid=(ni, nj),
        in_specs=[pl.BlockSpec((bm, d), lambda i, j: (i, 0)),
                  pl.BlockSpec((bn, d), lambda i, j: (j, 0))],
        out_specs=[pl.BlockSpec((1, 1, bm), lambda i, j: (i, 0, 0)),
                   pl.BlockSpec((1, 1, 128), lambda i, j: (i, 0, 0))],
        out_shape=[jax.ShapeDtypeStruct((ni, 1, bm), jnp.float32),
                   jax.ShapeDtypeStruct((ni, 1, 128), jnp.float32)],
        scratch_shapes=[pltpu.VMEM((bm, 128), jnp.float32)],
        compiler_params=pltpu.CompilerParams(
            dimension_semantics=("parallel", "arbitrary")),
        name="koleo_nn",
    )(xn, xn)

    del logd
    return -(jnp.sum(sums[:, 0, 0]) / n)


# speedup vs baseline: 9.7500x; 1.1091x over previous
import functools

import jax
import jax.numpy as jnp
from jax.experimental import pallas as pl
from jax.experimental.pallas import tpu as pltpu

_EPS = 1e-8


def _normalize_body(a_ref, b_ref, o_ref, *, f):
    a = a_ref[...]
    b = b_ref[...]
    ss = (jnp.sum(a * a, axis=1, keepdims=True)
          + jnp.sum(b * b, axis=1, keepdims=True))
    inv = 1.0 / jnp.maximum(jnp.sqrt(ss), _EPS)
    o_ref[:, :f] = (a * inv).astype(o_ref.dtype)
    o_ref[:, f:] = (b * inv).astype(o_ref.dtype)


def _nn_body(x_ref, y_ref, sum_ref, acc_ref, *, bm, bn, cn, nj, d):
    i = pl.program_id(0)
    j = pl.program_id(1)

    @pl.when(j == 0)
    def _():
        acc_ref[...] = jnp.full_like(acc_ref, -3.0)

    xi = x_ref[...]

    def accum(mask_diag):
        m = None
        for c in range(bn // cn):
            yc = y_ref[c * cn:(c + 1) * cn, :]
            sub = jax.lax.dot_general(
                xi, yc, (((1,), (1,)), ((), ())),
                preferred_element_type=jnp.float32)
            if mask_diag:
                rows = jax.lax.broadcasted_iota(jnp.int32, (bm, cn), 0)
                cols = jax.lax.broadcasted_iota(jnp.int32, (bm, cn), 1)
                sub = jnp.where(rows == cols + (j * bn + c * cn - i * bm),
                                -1.0, sub)
            for l in range(cn // 128):
                piece = sub[:, l * 128:(l + 1) * 128]
                m = piece if m is None else jnp.maximum(m, piece)
        acc_ref[...] = jnp.maximum(acc_ref[...], m)

    on_diag = (i * bm) // bn == j
    pl.when(on_diag)(lambda: accum(True))
    pl.when(jnp.logical_not(on_diag))(lambda: accum(False))

    @pl.when(j == nj - 1)
    def _():
        mrow = jnp.max(acc_ref[...], axis=1, keepdims=True)
        d2 = jnp.maximum(2.0 - 2.0 * mrow, 0.0)
        dist = jnp.sqrt(d2 + d * (_EPS * _EPS))
        logd = jnp.log(dist + _EPS)
        sum_ref[0, 0, :] = jnp.full((128,), jnp.sum(logd), jnp.float32)


def kernel(emg_latent, emg_parallel_latent):
    n, f = emg_latent.shape
    d = 2 * f

    rb = min(n, 2048)
    xn = pl.pallas_call(
        functools.partial(_normalize_body, f=f),
        grid=(n // rb,),
        in_specs=[pl.BlockSpec((rb, f), lambda r: (r, 0)),
                  pl.BlockSpec((rb, f), lambda r: (r, 0))],
        out_specs=pl.BlockSpec((rb, d), lambda r: (r, 0)),
        out_shape=jax.ShapeDtypeStruct((n, d), jnp.bfloat16),
        compiler_params=pltpu.CompilerParams(
            dimension_semantics=("parallel",)),
        name="koleo_normalize",
    )(emg_latent, emg_parallel_latent)

    bm = min(n, 1024)
    bn = min(n, 2048)
    cn = min(bn, 256)
    ni = n // bm
    nj = n // bn

    sums = pl.pallas_call(
        functools.partial(_nn_body, bm=bm, bn=bn, cn=cn, nj=nj, d=d),
        grid=(ni, nj),
        in_specs=[pl.BlockSpec((bm, d), lambda i, j: (i, 0)),
                  pl.BlockSpec((bn, d), lambda i, j: (j, 0))],
        out_specs=pl.BlockSpec((1, 1, 128), lambda i, j: (i, 0, 0)),
        out_shape=jax.ShapeDtypeStruct((ni, 1, 128), jnp.float32),
        scratch_shapes=[pltpu.VMEM((bm, 128), jnp.float32)],
        compiler_params=pltpu.CompilerParams(
            dimension_semantics=("parallel", "arbitrary")),
        name="koleo_nn",
    )(xn, xn)

    return -(jnp.sum(sums[:, 0, 0]) / n)


# speedup vs baseline: 10.2745x; 1.0538x over previous
import functools

import jax
import jax.numpy as jnp
from jax.experimental import pallas as pl
from jax.experimental.pallas import tpu as pltpu

_EPS = 1e-8


def _normalize_body(a_ref, b_ref, o_ref, *, f):
    a = a_ref[...]
    b = b_ref[...]
    ss = (jnp.sum(a * a, axis=1, keepdims=True)
          + jnp.sum(b * b, axis=1, keepdims=True))
    inv = 1.0 / jnp.maximum(jnp.sqrt(ss), _EPS)
    o_ref[:, :f] = (a * inv).astype(o_ref.dtype)
    o_ref[:, f:] = (b * inv).astype(o_ref.dtype)


def _nn_body(x_ref, y_ref, sum_ref, acc_ref, *, bm, bn, cn, nj, d):
    i = pl.program_id(0)
    j = pl.program_id(1)

    @pl.when(j == 0)
    def _():
        acc_ref[...] = jnp.full_like(acc_ref, -3.0)

    xi = x_ref[...]

    def accum(mask_diag):
        m = None
        for c in range(bn // cn):
            yc = y_ref[c * cn:(c + 1) * cn, :]
            sub = jax.lax.dot_general(
                xi, yc, (((1,), (1,)), ((), ())),
                preferred_element_type=jnp.float32)
            if mask_diag:
                rows = jax.lax.broadcasted_iota(jnp.int32, (bm, cn), 0)
                cols = jax.lax.broadcasted_iota(jnp.int32, (bm, cn), 1)
                sub = jnp.where(rows == cols + (j * bn + c * cn - i * bm),
                                -1.0, sub)
            for l in range(cn // 128):
                piece = sub[:, l * 128:(l + 1) * 128]
                m = piece if m is None else jnp.maximum(m, piece)
        acc_ref[...] = jnp.maximum(acc_ref[...], m)

    on_diag = (i * bm) // bn == j
    pl.when(on_diag)(lambda: accum(True))
    pl.when(jnp.logical_not(on_diag))(lambda: accum(False))

    @pl.when(j == nj - 1)
    def _():
        mrow = jnp.max(acc_ref[...], axis=1, keepdims=True)
        d2 = jnp.maximum(2.0 - 2.0 * mrow, 0.0)
        dist = jnp.sqrt(d2 + d * (_EPS * _EPS))
        logd = jnp.log(dist + _EPS)
        sum_ref[0, 0, :] = jnp.full((128,), jnp.sum(logd), jnp.float32)


def kernel(emg_latent, emg_parallel_latent):
    n, f = emg_latent.shape
    d = 2 * f

    rb = min(n, 2048)
    xn = pl.pallas_call(
        functools.partial(_normalize_body, f=f),
        grid=(n // rb,),
        in_specs=[pl.BlockSpec((rb, f), lambda r: (r, 0)),
                  pl.BlockSpec((rb, f), lambda r: (r, 0))],
        out_specs=pl.BlockSpec((rb, d), lambda r: (r, 0)),
        out_shape=jax.ShapeDtypeStruct((n, d), jnp.bfloat16),
        compiler_params=pltpu.CompilerParams(
            dimension_semantics=("parallel",)),
        name="koleo_normalize",
    )(emg_latent, emg_parallel_latent)

    bm = min(n, 1024)
    bn = min(n, 4096)
    cn = min(bn, 256)
    ni = n // bm
    nj = n // bn

    sums = pl.pallas_call(
        functools.partial(_nn_body, bm=bm, bn=bn, cn=cn, nj=nj, d=d),
        grid=(ni, nj),
        in_specs=[pl.BlockSpec((bm, d), lambda i, j: (i, 0)),
                  pl.BlockSpec((bn, d), lambda i, j: (j, 0))],
        out_specs=pl.BlockSpec((1, 1, 128), lambda i, j: (i, 0, 0)),
        out_shape=jax.ShapeDtypeStruct((ni, 1, 128), jnp.float32),
        scratch_shapes=[pltpu.VMEM((bm, 128), jnp.float32)],
        compiler_params=pltpu.CompilerParams(
            dimension_semantics=("parallel", "arbitrary")),
        name="koleo_nn",
    )(xn, xn)

    return -(jnp.sum(sums[:, 0, 0]) / n)


# speedup vs baseline: 10.5317x; 1.0250x over previous
import functools

import jax
import jax.numpy as jnp
from jax.experimental import pallas as pl
from jax.experimental.pallas import tpu as pltpu

_EPS = 1e-8


def _normalize_body(a_ref, b_ref, o_ref, *, f):
    a = a_ref[...]
    b = b_ref[...]
    ss = (jnp.sum(a * a, axis=1, keepdims=True)
          + jnp.sum(b * b, axis=1, keepdims=True))
    inv = 1.0 / jnp.maximum(jnp.sqrt(ss), _EPS)
    o_ref[:, :f] = (a * inv).astype(o_ref.dtype)
    o_ref[:, f:] = (b * inv).astype(o_ref.dtype)


def _nn_body(x_ref, y_ref, sum_ref, acc_ref, *, bm, bn, cn, nj, d):
    i = pl.program_id(0)
    j = pl.program_id(1)

    @pl.when(j == 0)
    def _():
        acc_ref[...] = jnp.full_like(acc_ref, -3.0)

    xi = x_ref[...]

    def accum(mask_diag):
        m = None
        for c in range(bn // cn):
            yc = y_ref[c * cn:(c + 1) * cn, :]
            sub = jax.lax.dot_general(
                xi, yc, (((1,), (1,)), ((), ())),
                preferred_element_type=jnp.float32)
            if mask_diag:
                rows = jax.lax.broadcasted_iota(jnp.int32, (bm, cn), 0)
                cols = jax.lax.broadcasted_iota(jnp.int32, (bm, cn), 1)
                sub = jnp.where(rows == cols + (j * bn + c * cn - i * bm),
                                -1.0, sub)
            for l in range(cn // 128):
                piece = sub[:, l * 128:(l + 1) * 128]
                m = piece if m is None else jnp.maximum(m, piece)
        acc_ref[...] = jnp.maximum(acc_ref[...], m)

    on_diag = (i * bm) // bn == j
    pl.when(on_diag)(lambda: accum(True))
    pl.when(jnp.logical_not(on_diag))(lambda: accum(False))

    @pl.when(j == nj - 1)
    def _():
        mrow = jnp.max(acc_ref[...], axis=1, keepdims=True)
        d2 = jnp.maximum(2.0 - 2.0 * mrow, 0.0)
        dist = jnp.sqrt(d2 + d * (_EPS * _EPS))
        logd = jnp.log(dist + _EPS)
        sum_ref[0, 0, :] = jnp.full((128,), jnp.sum(logd), jnp.float32)


def kernel(emg_latent, emg_parallel_latent):
    n, f = emg_latent.shape
    d = 2 * f

    rb = min(n, 2048)
    xn = pl.pallas_call(
        functools.partial(_normalize_body, f=f),
        grid=(n // rb,),
        in_specs=[pl.BlockSpec((rb, f), lambda r: (r, 0)),
                  pl.BlockSpec((rb, f), lambda r: (r, 0))],
        out_specs=pl.BlockSpec((rb, d), lambda r: (r, 0)),
        out_shape=jax.ShapeDtypeStruct((n, d), jnp.bfloat16),
        compiler_params=pltpu.CompilerParams(
            dimension_semantics=("parallel",)),
        name="koleo_normalize",
    )(emg_latent, emg_parallel_latent)

    bm = min(n, 1024)
    bn = min(n, 8192)
    cn = min(bn, 256)
    ni = n // bm
    nj = n // bn

    sums = pl.pallas_call(
        functools.partial(_nn_body, bm=bm, bn=bn, cn=cn, nj=nj, d=d),
        grid=(ni, nj),
        in_specs=[pl.BlockSpec((bm, d), lambda i, j: (i, 0)),
                  pl.BlockSpec((bn, d), lambda i, j: (j, 0))],
        out_specs=pl.BlockSpec((1, 1, 128), lambda i, j: (i, 0, 0)),
        out_shape=jax.ShapeDtypeStruct((ni, 1, 128), jnp.float32),
        scratch_shapes=[pltpu.VMEM((bm, 128), jnp.float32)],
        compiler_params=pltpu.CompilerParams(
            dimension_semantics=("parallel", "arbitrary")),
        name="koleo_nn",
    )(xn, xn)

    return -(jnp.sum(sums[:, 0, 0]) / n)
